# baseline (speedup 1.0000x reference)
import jax
import jax.numpy as jnp
from jax import lax
from jax.experimental import pallas as pl
from jax.experimental.pallas import tpu as pltpu

_KS = 4
_ST = 2
_PD = 1
_EPS = 1e-5
_B = 4


def _ceil_to(a, b):
    return (a + b - 1) // b * b


def kernel(x, weight, bias, gamma, beta):
    del bias
    N, C, H, W = x.shape
    Cout = weight.shape[0]
    Hout = (H + 2 * _PD - _KS) // _ST + 1
    Wout = (W + 2 * _PD - _KS) // _ST + 1
    Hc, Wc = Hout + 1, Wout + 1
    M = Hout * Wc
    Mc = Hout * Wout
    C4 = 4 * C
    Mp = _ceil_to(Hc * Wc + 2, 16)
    count = N * Mc
    G = N // _B

    xp = jnp.pad(x, ((0, 0), (0, 0), (_PD, _PD), (_PD, _PD)))
    xs = xp.reshape(N, C, Hc, 2, Wc, 2).transpose(0, 2, 4, 3, 5, 1)
    xs = xs.reshape(N, Hc * Wc, C4).astype(jnp.bfloat16)
    xs = jnp.pad(xs, ((0, 0), (0, Mp - Hc * Wc), (0, 0)))
    xs = xs.reshape(G, _B * Mp, C4)

    wt = weight.reshape(Cout, C, 2, 2, 2, 2)
    wt = wt.transpose(3, 5, 1, 2, 4, 0)
    wt = wt.reshape(C4, 4 * Cout).astype(jnp.bfloat16)

    offs = tuple(dh * Wc + dw for dh in range(2) for dw in range(2))

    def conv_kernel(xs_ref, w_ref, y_ref, sum_ref, ssq_ref):
        t = jnp.dot(xs_ref[0], w_ref[...], preferred_element_type=jnp.float32)
        s_acc = None
        q_acc = None
        for k in range(_B):
            b = k * Mp
            y = (t[b + offs[0]:b + offs[0] + M, :Cout]
                 + t[b + offs[1]:b + offs[1] + M, Cout:2 * Cout]
                 + t[b + offs[2]:b + offs[2] + M, 2 * Cout:3 * Cout]
                 + t[b + offs[3]:b + offs[3] + M, 3 * Cout:])
            yc = jnp.concatenate(
                [y[i * Wc:i * Wc + Wout] for i in range(Hout)], axis=0)
            s = jnp.sum(yc, axis=0, keepdims=True)
            q = jnp.sum(yc * yc, axis=0, keepdims=True)
            s_acc = s if s_acc is None else s_acc + s
            q_acc = q if q_acc is None else q_acc + q
            y_ref[0, k * Mc:(k + 1) * Mc, :] = yc.astype(jnp.bfloat16)
        sum_ref[0] = s_acc
        ssq_ref[0] = q_acc

    y, sums, ssqs = pl.pallas_call(
        conv_kernel,
        out_shape=(jax.ShapeDtypeStruct((G, _B * Mc, Cout), jnp.bfloat16),
                   jax.ShapeDtypeStruct((G, 1, Cout), jnp.float32),
                   jax.ShapeDtypeStruct((G, 1, Cout), jnp.float32)),
        grid=(G,),
        in_specs=[pl.BlockSpec((1, _B * Mp, C4), lambda n: (n, 0, 0)),
                  pl.BlockSpec((C4, 4 * Cout), lambda n: (0, 0))],
        out_specs=(pl.BlockSpec((1, _B * Mc, Cout), lambda n: (n, 0, 0)),
                   pl.BlockSpec((1, 1, Cout), lambda n: (n, 0, 0)),
                   pl.BlockSpec((1, 1, Cout), lambda n: (n, 0, 0))),
        compiler_params=pltpu.CompilerParams(
            dimension_semantics=("parallel",),
            vmem_limit_bytes=100 * 1024 * 1024),
    )(xs, wt)

    gamma2 = gamma.reshape(1, Cout)
    beta2 = beta.reshape(1, Cout)
    inv_count = float(1.0 / count)

    def norm_kernel(y_ref, sums_ref, ssqs_ref, gamma_ref, beta_ref, out_ref):
        mean = jnp.sum(sums_ref[:, 0, :], axis=0, keepdims=True) * inv_count
        msq = jnp.sum(ssqs_ref[:, 0, :], axis=0, keepdims=True) * inv_count
        var = jnp.maximum(msq - mean * mean, 0.0)
        scale = gamma_ref[...] * lax.rsqrt(var + _EPS)
        shift = beta_ref[...] - mean * scale
        z = jnp.maximum(y_ref[0].astype(jnp.float32) * scale + shift, 0.0)
        for k in range(_B):
            out_ref[0, k * Cout:(k + 1) * Cout, :] = z[k * Mc:(k + 1) * Mc, :].T

    out = pl.pallas_call(
        norm_kernel,
        out_shape=jax.ShapeDtypeStruct((G, _B * Cout, Mc), jnp.float32),
        grid=(G,),
        in_specs=[pl.BlockSpec((1, _B * Mc, Cout), lambda n: (n, 0, 0)),
                  pl.BlockSpec((G, 1, Cout), lambda n: (0, 0, 0)),
                  pl.BlockSpec((G, 1, Cout), lambda n: (0, 0, 0)),
                  pl.BlockSpec((1, Cout), lambda n: (0, 0)),
                  pl.BlockSpec((1, Cout), lambda n: (0, 0))],
        out_specs=pl.BlockSpec((1, _B * Cout, Mc), lambda n: (n, 0, 0)),
        compiler_params=pltpu.CompilerParams(
            dimension_semantics=("parallel",),
            vmem_limit_bytes=100 * 1024 * 1024),
    )(y, sums, ssqs, gamma2, beta2)

    return out.reshape(N, Cout, Hout, Wout)


# speedup vs baseline: 1.4346x; 1.4346x over previous
import jax
import jax.numpy as jnp
from jax import lax
from jax.experimental import pallas as pl
from jax.experimental.pallas import tpu as pltpu

_KS = 4
_ST = 2
_PD = 1
_EPS = 1e-5
_B = 4


def _ceil_to(a, b):
    return (a + b - 1) // b * b


def kernel(x, weight, bias, gamma, beta):
    del bias
    N, C, H, W = x.shape
    Cout = weight.shape[0]
    Hout = (H + 2 * _PD - _KS) // _ST + 1
    Wout = (W + 2 * _PD - _KS) // _ST + 1
    Hc, Wc = Hout + 1, Wout + 1
    M = Hout * Wc
    Mc = Hout * Wout
    C4 = 4 * C
    Mp = _ceil_to(Hc * Wc + 2, 16)
    count = N * Mc
    G = N // _B

    xp = jnp.pad(x.astype(jnp.bfloat16),
                 ((0, 0), (0, 0), (_PD, _PD), (_PD, _PD)))
    xs = xp.reshape(N, C, Hc, 2, Wc, 2).transpose(0, 2, 4, 3, 5, 1)
    xs = xs.reshape(G, _B * Hc * Wc, C4)

    wt = weight.reshape(Cout, C, 2, 2, 2, 2)
    wt = wt.transpose(3, 5, 1, 2, 4, 0)
    wt = wt.reshape(C4, 4 * Cout).astype(jnp.bfloat16)

    offs = tuple(dh * Wc + dw for dh in range(2) for dw in range(2))

    HW = Hc * Wc

    def conv_kernel(xs_ref, w_ref, y_ref, sum_ref, ssq_ref):
        t = jnp.dot(xs_ref[0], w_ref[...], preferred_element_type=jnp.float32)
        t = jnp.pad(t, ((0, 2), (0, 0)))
        s_acc = None
        q_acc = None
        for k in range(_B):
            b = k * HW
            y = (t[b + offs[0]:b + offs[0] + M, :Cout]
                 + t[b + offs[1]:b + offs[1] + M, Cout:2 * Cout]
                 + t[b + offs[2]:b + offs[2] + M, 2 * Cout:3 * Cout]
                 + t[b + offs[3]:b + offs[3] + M, 3 * Cout:])
            yc = jnp.concatenate(
                [y[i * Wc:i * Wc + Wout] for i in range(Hout)], axis=0)
            s = jnp.sum(yc, axis=0, keepdims=True)
            q = jnp.sum(yc * yc, axis=0, keepdims=True)
            s_acc = s if s_acc is None else s_acc + s
            q_acc = q if q_acc is None else q_acc + q
            y_ref[0, k * Mc:(k + 1) * Mc, :] = yc.astype(jnp.bfloat16)
        sum_ref[0] = s_acc
        ssq_ref[0] = q_acc

    y, sums, ssqs = pl.pallas_call(
        conv_kernel,
        out_shape=(jax.ShapeDtypeStruct((G, _B * Mc, Cout), jnp.bfloat16),
                   jax.ShapeDtypeStruct((G, 1, Cout), jnp.float32),
                   jax.ShapeDtypeStruct((G, 1, Cout), jnp.float32)),
        grid=(G,),
        in_specs=[pl.BlockSpec((1, _B * HW, C4), lambda n: (n, 0, 0)),
                  pl.BlockSpec((C4, 4 * Cout), lambda n: (0, 0))],
        out_specs=(pl.BlockSpec((1, _B * Mc, Cout), lambda n: (n, 0, 0)),
                   pl.BlockSpec((1, 1, Cout), lambda n: (n, 0, 0)),
                   pl.BlockSpec((1, 1, Cout), lambda n: (n, 0, 0))),
        compiler_params=pltpu.CompilerParams(
            dimension_semantics=("parallel",),
            vmem_limit_bytes=100 * 1024 * 1024),
    )(xs, wt)

    gamma2 = gamma.reshape(1, Cout)
    beta2 = beta.reshape(1, Cout)
    inv_count = float(1.0 / count)

    def norm_kernel(y_ref, sums_ref, ssqs_ref, gamma_ref, beta_ref, out_ref):
        mean = jnp.sum(sums_ref[:, 0, :], axis=0, keepdims=True) * inv_count
        msq = jnp.sum(ssqs_ref[:, 0, :], axis=0, keepdims=True) * inv_count
        var = jnp.maximum(msq - mean * mean, 0.0)
        scale = gamma_ref[...] * lax.rsqrt(var + _EPS)
        shift = beta_ref[...] - mean * scale
        z = jnp.maximum(y_ref[0].astype(jnp.float32) * scale + shift, 0.0)
        out_ref[0] = z

    out = pl.pallas_call(
        norm_kernel,
        out_shape=jax.ShapeDtypeStruct((G, _B * Mc, Cout), jnp.float32),
        grid=(G,),
        in_specs=[pl.BlockSpec((1, _B * Mc, Cout), lambda n: (n, 0, 0)),
                  pl.BlockSpec((G, 1, Cout), lambda n: (0, 0, 0)),
                  pl.BlockSpec((G, 1, Cout), lambda n: (0, 0, 0)),
                  pl.BlockSpec((1, Cout), lambda n: (0, 0)),
                  pl.BlockSpec((1, Cout), lambda n: (0, 0))],
        out_specs=pl.BlockSpec((1, _B * Mc, Cout), lambda n: (n, 0, 0)),
        compiler_params=pltpu.CompilerParams(
            dimension_semantics=("parallel",),
            vmem_limit_bytes=100 * 1024 * 1024),
    )(y, sums, ssqs, gamma2, beta2)

    return out.reshape(N, Hout, Wout, Cout).transpose(0, 3, 1, 2)
